# baseline (speedup 1.0000x reference)
import jax
import jax.numpy as jnp
from jax.experimental import pallas as pl
from jax.experimental.pallas import tpu as pltpu

_P = 32
_CH = 8
_LOG2E = 1.4426950408889634
_LN2 = 0.6931471805599453


def _loss_kernel(eps0_ref, eps1_ref, aux_ref, out_ref):
    t = eps0_ref.shape[1]
    nb = eps0_ref.shape[2]
    steps = t // _CH
    rem = t - steps * _CH

    def slab(eps_ref, base, off, rows, accs):
        l0 = aux_ref[base + 0:base + 1, :]
        l1 = aux_ref[base + 1:base + 2, :]
        l2 = aux_ref[base + 2:base + 3, :]
        s2 = aux_ref[base + 3:base + 4, :]
        x0 = eps_ref[0, pl.ds(off, rows), :]
        x1 = eps_ref[1, pl.ds(off, rows), :]
        x2 = eps_ref[2, pl.ds(off, rows), :]
        e = (jnp.exp2(l0 + x0 * s2) + jnp.exp2(l1 + x1 * s2)
             + jnp.exp2(l2 + x2 * s2))
        lg = jnp.log2(jnp.maximum(e, 1e-30))
        aL, aX0, aX1, aX2 = accs
        return aL + lg, aX0 + x0, aX1 + x1, aX2 + x2

    def body(i, carry):
        a0, a1 = carry
        off = pl.multiple_of(i * _CH, _CH)
        return slab(eps0_ref, 0, off, _CH, a0), slab(eps1_ref, 12, off, _CH, a1)

    zeros = jnp.zeros((_CH, nb), jnp.float32)
    init = ((zeros, zeros, zeros, zeros), (zeros, zeros, zeros, zeros))
    acc0, acc1 = jax.lax.fori_loop(0, steps, body, init)

    def finalize(eps_ref, base, accs):
        aL, aX0, aX1, aX2 = (jnp.sum(a, axis=0, keepdims=True) for a in accs)
        if rem:
            eL, eX0, eX1, eX2 = slab(
                eps_ref, base, steps * _CH, rem,
                tuple(jnp.zeros((rem, nb), jnp.float32) for _ in range(4)))
            aL = aL + jnp.sum(eL, axis=0, keepdims=True)
            aX0 = aX0 + jnp.sum(eX0, axis=0, keepdims=True)
            aX1 = aX1 + jnp.sum(eX1, axis=0, keepdims=True)
            aX2 = aX2 + jnp.sum(eX2, axis=0, keepdims=True)
        w0 = aux_ref[base + 4:base + 5, :]
        w1 = aux_ref[base + 5:base + 6, :]
        w2 = aux_ref[base + 6:base + 7, :]
        yt = aux_ref[base + 7:base + 8, :]
        sc = aux_ref[base + 8:base + 9, :]
        tdotwl = aux_ref[base + 9:base + 10, :]
        return (yt * (_LN2 * aL) - tdotwl
                - sc * (w0 * aX0 + w1 * aX1 + w2 * aX2))

    out_ref[0] = jnp.concatenate(
        [finalize(eps0_ref, 0, acc0), finalize(eps1_ref, 12, acc1)], axis=0)


def _aux_rows(y_true, y_pred, t):
    lg = y_pred[:, :3].T
    sc = jnp.exp(0.5 * y_pred[:, 3])[None, :]
    w = y_true.T
    yt = jnp.sum(y_true, axis=1)[None, :]
    tdotwl = t * jnp.sum(w * lg, axis=0, keepdims=True)
    z = jnp.zeros_like(sc)
    return jnp.concatenate(
        [lg * _LOG2E, sc * _LOG2E, w, yt, sc, tdotwl, z, z], axis=0)


def kernel(y_true0, y_pred0, y_true1, y_pred1, log_vars, eps0, eps1):
    t, n, _ = eps0.shape
    nb = n // _P

    e0 = jnp.transpose(eps0, (2, 0, 1))
    e1 = jnp.transpose(eps1, (2, 0, 1))
    aux = jnp.concatenate(
        [_aux_rows(y_true0, y_pred0, t), _aux_rows(y_true1, y_pred1, t)],
        axis=0)

    out = pl.pallas_call(
        _loss_kernel,
        grid=(_P,),
        in_specs=[
            pl.BlockSpec((3, t, nb), lambda p: (0, 0, p)),
            pl.BlockSpec((3, t, nb), lambda p: (0, 0, p)),
            pl.BlockSpec((24, nb), lambda p: (0, p)),
        ],
        out_specs=pl.BlockSpec((1, 2, nb), lambda p: (p, 0, 0)),
        out_shape=jax.ShapeDtypeStruct((_P, 2, nb), jnp.float32),
        compiler_params=pltpu.CompilerParams(
            dimension_semantics=("parallel",)),
    )(e0, e1, aux)

    inv_tn = 1.0 / (t * n)
    mc0 = jnp.sum(out[:, 0, :]) * inv_tn
    mc1 = jnp.sum(out[:, 1, :]) * inv_tn
    lv0, lv1 = log_vars[0], log_vars[1]
    return jnp.exp(-lv0) * mc0 + lv0 + jnp.exp(-lv1) * mc1 + lv1
